# baseline (speedup 1.0000x reference)
import functools

import jax
import jax.numpy as jnp
from jax import lax
from jax.experimental import pallas as pl
from jax.experimental.pallas import tpu as pltpu
from jax.experimental.pallas import tpu_sc as plsc

_VOCAB = 1000000
_EMB = 64
_B = 4096
_L = 200

_FOLD = 8
_TVN = _VOCAB // _FOLD
_TVK = _EMB * _FOLD
_TV_BLK = 5000

_NC, _NS = 2, 16
_NW = _NC * _NS
_DPW = _B // _NW


def _tv_body(w_ref, t_ref, o_ref):
    dot = lax.dot_general(t_ref[...], w_ref[...], (((1,), (0,)), ((), ())),
                          preferred_element_type=jnp.float32)
    gid = lax.broadcasted_iota(jnp.int32, (_TV_BLK, _FOLD), 0) * _FOLD \
        + lax.broadcasted_iota(jnp.int32, (_TV_BLK, _FOLD), 1) \
        + pl.program_id(0) * (_TV_BLK * _FOLD)
    o_ref[...] = jnp.where(gid == 0, 0.0, dot)


def _compute_tv(table8, w8):
    return pl.pallas_call(
        _tv_body,
        grid=(_TVN // _TV_BLK,),
        in_specs=[
            pl.BlockSpec((_TVK, _FOLD), lambda i: (0, 0)),
            pl.BlockSpec((_TV_BLK, _TVK), lambda i: (i, 0)),
        ],
        out_specs=pl.BlockSpec((_TV_BLK, _FOLD), lambda i: (i, 0)),
        out_shape=jax.ShapeDtypeStruct((_TVN, _FOLD), jnp.float32),
    )(w8, table8)


_IPW = _L * _DPW


@functools.partial(
    pl.kernel,
    out_type=jax.ShapeDtypeStruct((_B,), jnp.float32),
    mesh=plsc.VectorSubcoreMesh(core_axis_name="c", subcore_axis_name="s"),
    scratch_types=[
        pltpu.VMEM((_IPW,), jnp.int32),
        pltpu.VMEM((_IPW,), jnp.float32),
        pltpu.VMEM((_DPW,), jnp.float32),
        pltpu.VMEM((16,), jnp.float32),
        pltpu.SemaphoreType.DMA,
    ],
)
def _sc_pool(xw_hbm, tv_hbm, b_hbm, out_hbm, idx_v, vals_v, out_v, b_v, sem):
    wid = lax.axis_index("s") * _NC + lax.axis_index("c")
    base = wid * _DPW
    pltpu.sync_copy(b_hbm, b_v)
    pltpu.sync_copy(xw_hbm.at[wid], idx_v)
    pltpu.async_copy(tv_hbm.at[idx_v], vals_v, sem).wait()
    bias = b_v[...]
    nchunk = _DPW // 16

    def body(r, accs):
        off = r * _DPW
        return tuple(a + vals_v[pl.ds(off + 16 * c, 16)]
                     for c, a in enumerate(accs))

    accs = lax.fori_loop(0, _L, body, (bias,) * nchunk)
    for c in range(nchunk):
        out_v[pl.ds(c * 16, 16)] = accs[c]
    pltpu.sync_copy(out_v, out_hbm.at[pl.ds(base, _DPW)])


def kernel(x, table, W, b):
    x = x.astype(jnp.int32)
    xw = (x.reshape(_NW, _DPW, _L)
           .transpose(0, 2, 1)
           .reshape(_NW, _IPW))
    b16 = jnp.broadcast_to(b.astype(jnp.float32), (16,))
    w8 = jnp.kron(jnp.eye(_FOLD, dtype=jnp.float32),
                  W.astype(jnp.float32) * (1.0 / _L)).T
    table8 = table.reshape(_TVN, _TVK)
    tv = _compute_tv(table8, w8).reshape(_VOCAB)
    out = _sc_pool(xw, tv, b16)
    return out.reshape(_B, 1)
